# speedup vs baseline: 7.6053x; 7.6053x over previous
import functools

import jax
import jax.numpy as jnp
from jax import lax
from jax.experimental import pallas as pl
from jax.experimental.pallas import tpu as pltpu
from jax.experimental.pallas import tpu_sc as plsc

VOCAB = 1300
D = 128
BATCH = 16384
HIST = 200
B = BATCH * HIST

NC, NS = 2, 16
NW = NC * NS
BPW = B // NW
C = 400
NCHUNK = BPW // C

_MESH = plsc.VectorSubcoreMesh(core_axis_name="c", subcore_axis_name="s")


@functools.partial(
    pl.kernel,
    out_type=jax.ShapeDtypeStruct((B, D), jnp.float32),
    mesh=_MESH,
    scratch_types=[
        pltpu.VMEM((C,), jnp.int32),
        pltpu.VMEM((C, D), jnp.float32),
        pltpu.SemaphoreType.DMA,
    ],
)
def _sc_gather(idx_hbm, table_hbm, out_hbm, idx_v, rows_v, sem):
    wid = lax.axis_index("s") * NC + lax.axis_index("c")
    base = wid * BPW

    def chunk(i, carry):
        off = base + i * C
        pltpu.sync_copy(idx_hbm.at[pl.ds(off, C)], idx_v)
        pltpu.async_copy(table_hbm.at[idx_v], rows_v, sem).wait()
        pltpu.sync_copy(rows_v, out_hbm.at[pl.ds(off, C)])
        return carry

    lax.fori_loop(0, NCHUNK, chunk, 0)


def kernel(indices, species):
    idx = indices.reshape(-1).astype(jnp.int32)
    out = _sc_gather(idx, species)
    return out.reshape(BATCH, HIST, D)


# speedup vs baseline: 7.7773x; 1.0226x over previous
import functools

import jax
import jax.numpy as jnp
from jax import lax
from jax.experimental import pallas as pl
from jax.experimental.pallas import tpu as pltpu
from jax.experimental.pallas import tpu_sc as plsc

VOCAB = 1300
D = 128
BATCH = 16384
HIST = 200
B = BATCH * HIST

NC, NS = 2, 16
NW = NC * NS
C = 256
NCHUNK = B // (NW * C)
SUP = 16
NSUP = NCHUNK // SUP

_MESH = plsc.VectorSubcoreMesh(core_axis_name="c", subcore_axis_name="s")


@functools.partial(
    pl.kernel,
    out_type=jax.ShapeDtypeStruct((B, D), jnp.float32),
    mesh=_MESH,
    scratch_types=[
        pltpu.VMEM((SUP * C,), jnp.int32),
        pltpu.VMEM((C, D), jnp.float32),
        pltpu.VMEM((C, D), jnp.float32),
        pltpu.SemaphoreType.DMA,
        pltpu.SemaphoreType.DMA,
        pltpu.SemaphoreType.DMA,
    ],
)
def _sc_gather(idx_hbm, table_hbm, out_hbm, idx_v, rows0, rows1, gsem, ssem0, ssem1):
    wid = lax.axis_index("s") * NC + lax.axis_index("c")
    chunk0 = wid * NCHUNK

    def superchunk(s, _):
        pltpu.sync_copy(
            idx_hbm.at[pl.ds((chunk0 + s * SUP) * C, SUP * C)], idx_v
        )

        def pair(g, _):
            for b, rows_v, ssem in ((0, rows0, ssem0), (1, rows1, ssem1)):
                j = 2 * g + b
                i = s * SUP + j
                row = (chunk0 + i) * C

                @pl.when(i >= 2)
                def _wait_store():
                    pltpu.make_async_copy(
                        rows_v, out_hbm.at[pl.ds(0, C)], ssem
                    ).wait()

                pltpu.async_copy(
                    table_hbm.at[idx_v.at[pl.ds(j * C, C)]], rows_v, gsem
                ).wait()
                pltpu.async_copy(rows_v, out_hbm.at[pl.ds(row, C)], ssem)
            return 0

        lax.fori_loop(0, SUP // 2, pair, 0)
        return 0

    lax.fori_loop(0, NSUP, superchunk, 0)
    pltpu.make_async_copy(rows0, out_hbm.at[pl.ds(0, C)], ssem0).wait()
    pltpu.make_async_copy(rows1, out_hbm.at[pl.ds(0, C)], ssem1).wait()


def kernel(indices, species):
    idx = indices.reshape(-1).astype(jnp.int32)
    out = _sc_gather(idx, species)
    return out.reshape(BATCH, HIST, D)


# speedup vs baseline: 7.8060x; 1.0037x over previous
import functools

import jax
import jax.numpy as jnp
from jax import lax
from jax.experimental import pallas as pl
from jax.experimental.pallas import tpu as pltpu
from jax.experimental.pallas import tpu_sc as plsc

VOCAB = 1300
D = 128
BATCH = 16384
HIST = 200
B = BATCH * HIST

NC, NS = 2, 16
NW = NC * NS
C = 256
NCHUNK = B // (NW * C)
SUP = 16
NSUP = NCHUNK // SUP

_MESH = plsc.VectorSubcoreMesh(core_axis_name="c", subcore_axis_name="s")


@functools.partial(
    pl.kernel,
    out_type=jax.ShapeDtypeStruct((B, D), jnp.float32),
    mesh=_MESH,
    scratch_types=[
        pltpu.VMEM((SUP * C,), jnp.int32),
        pltpu.VMEM((C, D), jnp.float32),
        pltpu.VMEM((C, D), jnp.float32),
        pltpu.SemaphoreType.DMA,
        pltpu.SemaphoreType.DMA,
        pltpu.SemaphoreType.DMA,
        pltpu.SemaphoreType.DMA,
    ],
)
def _sc_gather(
    idx_hbm, table_hbm, out_hbm, idx_v, rows0, rows1, gsem0, gsem1, ssem0, ssem1
):
    wid = lax.axis_index("s") * NC + lax.axis_index("c")
    chunk0 = wid * NCHUNK

    def start_gather(j, rows_v, gsem):
        pltpu.async_copy(table_hbm.at[idx_v.at[pl.ds(j * C, C)]], rows_v, gsem)

    def wait_gather(rows_v, gsem):
        pltpu.make_async_copy(
            table_hbm.at[idx_v.at[pl.ds(0, C)]], rows_v, gsem
        ).wait()

    def start_store(row, rows_v, ssem):
        pltpu.async_copy(rows_v, out_hbm.at[pl.ds(row, C)], ssem)

    def wait_store(rows_v, ssem):
        pltpu.make_async_copy(rows_v, out_hbm.at[pl.ds(0, C)], ssem).wait()

    def superchunk(s, _):
        sup_row = (chunk0 + s * SUP) * C

        pltpu.sync_copy(idx_hbm.at[pl.ds(sup_row, SUP * C)], idx_v)

        @pl.when(s > 0)
        def _():
            wait_store(rows0, ssem0)

        start_gather(0, rows0, gsem0)

        def pair(g, _):
            @pl.when((s > 0) | (g > 0))
            def _():
                wait_store(rows1, ssem1)

            start_gather(2 * g + 1, rows1, gsem1)
            wait_gather(rows0, gsem0)
            start_store(sup_row + 2 * g * C, rows0, ssem0)

            @pl.when(g < SUP // 2 - 1)
            def _():
                wait_store(rows0, ssem0)
                start_gather(2 * g + 2, rows0, gsem0)

            wait_gather(rows1, gsem1)
            start_store(sup_row + (2 * g + 1) * C, rows1, ssem1)
            return 0

        lax.fori_loop(0, SUP // 2, pair, 0)
        return 0

    lax.fori_loop(0, NSUP, superchunk, 0)
    wait_store(rows0, ssem0)
    wait_store(rows1, ssem1)


def kernel(indices, species):
    idx = indices.reshape(-1).astype(jnp.int32)
    out = _sc_gather(idx, species)
    return out.reshape(BATCH, HIST, D)


# speedup vs baseline: 18.5137x; 2.3717x over previous
import functools

import jax
import jax.numpy as jnp
from jax import lax
from jax.experimental import pallas as pl
from jax.experimental.pallas import tpu as pltpu
from jax.experimental.pallas import tpu_sc as plsc

VOCAB = 1300
D = 128
BATCH = 16384
HIST = 200
B = BATCH * HIST

NC, NS = 2, 16
NW = NC * NS
C = 256
NCHUNK = B // (NW * C)
SUP = 16
NSUP = NCHUNK // SUP

_MESH = plsc.VectorSubcoreMesh(core_axis_name="c", subcore_axis_name="s")


@functools.partial(
    pl.kernel,
    out_type=jax.ShapeDtypeStruct((B, D), jnp.float32),
    mesh=_MESH,
    scratch_types=[
        pltpu.VMEM((SUP * C,), jnp.int32),
        pltpu.VMEM((C, D), jnp.float32),
        pltpu.VMEM((C, D), jnp.float32),
        pltpu.VMEM_SHARED((VOCAB, D), jnp.float32),
        pltpu.SemaphoreType.DMA,
        pltpu.SemaphoreType.DMA,
        pltpu.SemaphoreType.DMA,
        pltpu.SemaphoreType.DMA,
    ],
)
def _sc_gather(
    idx_hbm,
    table_hbm,
    out_hbm,
    idx_v,
    rows0,
    rows1,
    table_sh,
    gsem0,
    gsem1,
    ssem0,
    ssem1,
):
    wid = lax.axis_index("s") * NC + lax.axis_index("c")
    chunk0 = wid * NCHUNK

    @pl.when(lax.axis_index("s") == 0)
    def _():
        pltpu.sync_copy(table_hbm, table_sh)

    plsc.subcore_barrier()

    def start_gather(j, rows_v, gsem):
        pltpu.async_copy(table_sh.at[idx_v.at[pl.ds(j * C, C)]], rows_v, gsem)

    def wait_gather(rows_v, gsem):
        pltpu.make_async_copy(
            table_sh.at[idx_v.at[pl.ds(0, C)]], rows_v, gsem
        ).wait()

    def start_store(row, rows_v, ssem):
        pltpu.async_copy(rows_v, out_hbm.at[pl.ds(row, C)], ssem)

    def wait_store(rows_v, ssem):
        pltpu.make_async_copy(rows_v, out_hbm.at[pl.ds(0, C)], ssem).wait()

    def superchunk(s, _):
        sup_row = (chunk0 + s * SUP) * C

        pltpu.sync_copy(idx_hbm.at[pl.ds(sup_row, SUP * C)], idx_v)

        @pl.when(s > 0)
        def _():
            wait_store(rows0, ssem0)

        start_gather(0, rows0, gsem0)

        def pair(g, _):
            @pl.when((s > 0) | (g > 0))
            def _():
                wait_store(rows1, ssem1)

            start_gather(2 * g + 1, rows1, gsem1)
            wait_gather(rows0, gsem0)
            start_store(sup_row + 2 * g * C, rows0, ssem0)

            @pl.when(g < SUP // 2 - 1)
            def _():
                wait_store(rows0, ssem0)
                start_gather(2 * g + 2, rows0, gsem0)

            wait_gather(rows1, gsem1)
            start_store(sup_row + (2 * g + 1) * C, rows1, ssem1)
            return 0

        lax.fori_loop(0, SUP // 2, pair, 0)
        return 0

    lax.fori_loop(0, NSUP, superchunk, 0)
    wait_store(rows0, ssem0)
    wait_store(rows1, ssem1)


def kernel(indices, species):
    idx = indices.reshape(-1).astype(jnp.int32)
    out = _sc_gather(idx, species)
    return out.reshape(BATCH, HIST, D)
